# Initial kernel scaffold; baseline (speedup 1.0000x reference)
import jax
import jax.numpy as jnp
from jax.experimental import pallas as pl


def kernel(x):
    raise NotImplementedError("write your pallas kernel here")


# speedup vs baseline: 2.1502x; 2.1502x over previous
import jax
import jax.numpy as jnp
from jax import lax
from jax.experimental import pallas as pl
from jax.experimental.pallas import tpu as pltpu

_R = 3
_K = 2 * _R + 1


def _box_kernel(x_ref, o_ref):
    z = x_ref[0]
    cb, h, w = z.shape
    e = jnp.exp(z)
    d = 0.5 * jnp.exp(z * z - z)

    zpad_w = jnp.zeros((cb, h, _R), jnp.float32)
    p = jnp.concatenate([zpad_w, d, zpad_w], axis=2)
    s1 = p[:, :, 0:w]
    for k in range(1, _K):
        s1 = s1 + p[:, :, k:k + w]

    zpad_h = jnp.zeros((cb, _R, w), jnp.float32)
    q = jnp.concatenate([zpad_h, s1, zpad_h], axis=1)
    s2 = q[:, 0:h, :]
    for k in range(1, _K):
        s2 = s2 + q[:, k:k + h, :]

    ir = lax.broadcasted_iota(jnp.int32, (h, w), 0).astype(jnp.float32)
    ic = lax.broadcasted_iota(jnp.int32, (h, w), 1).astype(jnp.float32)
    fr = _R * 1.0
    cnt_r = jnp.minimum(ir, fr) + jnp.minimum((h - 1) - ir, fr) + 1.0
    cnt_c = jnp.minimum(ic, fr) + jnp.minimum((w - 1) - ic, fr) + 1.0
    inv_cnt = 1.0 / (cnt_r * cnt_c)

    o_ref[0, 0] = z
    o_ref[0, 1] = s2 * inv_cnt[None, :, :] * e


def kernel(x):
    b, c, h, w = x.shape
    cb = 8
    grid = (b, c // cb)
    out = pl.pallas_call(
        _box_kernel,
        out_shape=jax.ShapeDtypeStruct((b, 2, c, h, w), x.dtype),
        grid=grid,
        in_specs=[pl.BlockSpec((1, cb, h, w), lambda i, j: (i, j, 0, 0))],
        out_specs=pl.BlockSpec((1, 2, cb, h, w), lambda i, j: (i, 0, j, 0, 0)),
        compiler_params=pltpu.CompilerParams(
            dimension_semantics=("parallel", "parallel"),
        ),
        name="region_gaussian_fused",
    )(x)
    return out.reshape(b, 2 * c, h, w)


# speedup vs baseline: 7.2246x; 3.3600x over previous
import jax
import jax.numpy as jnp
from jax import lax
from jax.experimental import pallas as pl
from jax.experimental.pallas import tpu as pltpu

_R = 3
_K = 2 * _R + 1


def _box_kernel(x_ref, bh_ref, bw_ref, inv_ref, o_ref):
    cb = x_ref.shape[1]
    bh = bh_ref[...]
    bw = bw_ref[...]
    inv_cnt = inv_ref[...]

    for c in range(cb):
        z = x_ref[0, c]
        e = jnp.exp(z)
        d = 0.5 * jnp.exp(z * z - z)
        u = jnp.dot(d.astype(jnp.bfloat16), bw,
                    preferred_element_type=jnp.float32)
        s2 = jnp.dot(bh, u.astype(jnp.bfloat16),
                     preferred_element_type=jnp.float32)
        o_ref[0, 0, c] = z
        o_ref[0, 1, c] = s2 * inv_cnt * e


def kernel(x):
    b, c, h, w = x.shape
    cb = 8

    ih = lax.broadcasted_iota(jnp.int32, (h, h), 0)
    band_h = (jnp.abs(ih - ih.T) <= _R).astype(jnp.bfloat16)
    iw = lax.broadcasted_iota(jnp.int32, (w, w), 0)
    band_w = (jnp.abs(iw - iw.T) <= _R).astype(jnp.bfloat16)

    ir = lax.broadcasted_iota(jnp.float32, (h, 1), 0)
    ic = lax.broadcasted_iota(jnp.float32, (1, w), 1)
    fr = float(_R)
    cnt_r = jnp.minimum(ir, fr) + jnp.minimum((h - 1) - ir, fr) + 1.0
    cnt_c = jnp.minimum(ic, fr) + jnp.minimum((w - 1) - ic, fr) + 1.0
    inv_cnt = 1.0 / (cnt_r * cnt_c)

    grid = (b, c // cb)
    out = pl.pallas_call(
        _box_kernel,
        out_shape=jax.ShapeDtypeStruct((b, 2, c, h, w), x.dtype),
        grid=grid,
        in_specs=[
            pl.BlockSpec((1, cb, h, w), lambda i, j: (i, j, 0, 0)),
            pl.BlockSpec((h, h), lambda i, j: (0, 0)),
            pl.BlockSpec((w, w), lambda i, j: (0, 0)),
            pl.BlockSpec((h, w), lambda i, j: (0, 0)),
        ],
        out_specs=pl.BlockSpec((1, 2, cb, h, w), lambda i, j: (i, 0, j, 0, 0)),
        compiler_params=pltpu.CompilerParams(
            dimension_semantics=("parallel", "parallel"),
        ),
        name="region_gaussian_fused",
    )(x, band_h, band_w, inv_cnt)
    return out.reshape(b, 2 * c, h, w)


# speedup vs baseline: 7.7839x; 1.0774x over previous
import jax
import jax.numpy as jnp
from jax import lax
from jax.experimental import pallas as pl
from jax.experimental.pallas import tpu as pltpu

_R = 3
_K = 2 * _R + 1


def _box_kernel(x_ref, bh_ref, bw_ref, inv_ref, o_ref):
    cb = x_ref.shape[1]
    bh = bh_ref[...]
    bw = bw_ref[...]
    inv_cnt = inv_ref[...]

    for c in range(cb):
        z = x_ref[0, c]
        e = jnp.exp(z)
        d = 0.5 * jnp.exp(z * z - z)
        u = jnp.dot(d.astype(jnp.bfloat16), bw,
                    preferred_element_type=jnp.float32)
        s2 = jnp.dot(bh, u.astype(jnp.bfloat16),
                     preferred_element_type=jnp.float32)
        o_ref[0, 0, c] = z
        o_ref[0, 1, c] = s2 * inv_cnt * e


def kernel(x):
    b, c, h, w = x.shape
    cb = 16

    ih = lax.broadcasted_iota(jnp.int32, (h, h), 0)
    band_h = (jnp.abs(ih - ih.T) <= _R).astype(jnp.bfloat16)
    iw = lax.broadcasted_iota(jnp.int32, (w, w), 0)
    band_w = (jnp.abs(iw - iw.T) <= _R).astype(jnp.bfloat16)

    ir = lax.broadcasted_iota(jnp.float32, (h, 1), 0)
    ic = lax.broadcasted_iota(jnp.float32, (1, w), 1)
    fr = float(_R)
    cnt_r = jnp.minimum(ir, fr) + jnp.minimum((h - 1) - ir, fr) + 1.0
    cnt_c = jnp.minimum(ic, fr) + jnp.minimum((w - 1) - ic, fr) + 1.0
    inv_cnt = 1.0 / (cnt_r * cnt_c)

    grid = (b, c // cb)
    out = pl.pallas_call(
        _box_kernel,
        out_shape=jax.ShapeDtypeStruct((b, 2, c, h, w), x.dtype),
        grid=grid,
        in_specs=[
            pl.BlockSpec((1, cb, h, w), lambda i, j: (i, j, 0, 0)),
            pl.BlockSpec((h, h), lambda i, j: (0, 0)),
            pl.BlockSpec((w, w), lambda i, j: (0, 0)),
            pl.BlockSpec((h, w), lambda i, j: (0, 0)),
        ],
        out_specs=pl.BlockSpec((1, 2, cb, h, w), lambda i, j: (i, 0, j, 0, 0)),
        compiler_params=pltpu.CompilerParams(
            dimension_semantics=("parallel", "parallel"),
        ),
        name="region_gaussian_fused",
    )(x, band_h, band_w, inv_cnt)
    return out.reshape(b, 2 * c, h, w)


# speedup vs baseline: 8.0121x; 1.0293x over previous
import jax
import jax.numpy as jnp
from jax import lax
from jax.experimental import pallas as pl
from jax.experimental.pallas import tpu as pltpu

_R = 3
_K = 2 * _R + 1


def _box_kernel(x_ref, bh_ref, bw_ref, inv_ref, o_ref):
    cb = x_ref.shape[1]
    bh = bh_ref[...]
    bw = bw_ref[...]
    inv_cnt = inv_ref[...]

    for c in range(cb):
        z = x_ref[0, c]
        e = jnp.exp(z)
        d = 0.5 * jnp.exp(z * z - z)
        u = jnp.dot(d.astype(jnp.bfloat16), bw,
                    preferred_element_type=jnp.float32)
        s2 = jnp.dot(bh, u.astype(jnp.bfloat16),
                     preferred_element_type=jnp.float32)
        o_ref[0, 0, c] = z
        o_ref[0, 1, c] = s2 * inv_cnt * e


def kernel(x):
    b, c, h, w = x.shape
    cb = 32

    ih = lax.broadcasted_iota(jnp.int32, (h, h), 0)
    band_h = (jnp.abs(ih - ih.T) <= _R).astype(jnp.bfloat16)
    iw = lax.broadcasted_iota(jnp.int32, (w, w), 0)
    band_w = (jnp.abs(iw - iw.T) <= _R).astype(jnp.bfloat16)

    ir = lax.broadcasted_iota(jnp.float32, (h, 1), 0)
    ic = lax.broadcasted_iota(jnp.float32, (1, w), 1)
    fr = float(_R)
    cnt_r = jnp.minimum(ir, fr) + jnp.minimum((h - 1) - ir, fr) + 1.0
    cnt_c = jnp.minimum(ic, fr) + jnp.minimum((w - 1) - ic, fr) + 1.0
    inv_cnt = 1.0 / (cnt_r * cnt_c)

    grid = (b, c // cb)
    out = pl.pallas_call(
        _box_kernel,
        out_shape=jax.ShapeDtypeStruct((b, 2, c, h, w), x.dtype),
        grid=grid,
        in_specs=[
            pl.BlockSpec((1, cb, h, w), lambda i, j: (i, j, 0, 0)),
            pl.BlockSpec((h, h), lambda i, j: (0, 0)),
            pl.BlockSpec((w, w), lambda i, j: (0, 0)),
            pl.BlockSpec((h, w), lambda i, j: (0, 0)),
        ],
        out_specs=pl.BlockSpec((1, 2, cb, h, w), lambda i, j: (i, 0, j, 0, 0)),
        compiler_params=pltpu.CompilerParams(
            dimension_semantics=("parallel", "parallel"),
        ),
        name="region_gaussian_fused",
    )(x, band_h, band_w, inv_cnt)
    return out.reshape(b, 2 * c, h, w)
